# Initial kernel scaffold; baseline (speedup 1.0000x reference)
import jax
import jax.numpy as jnp
from jax.experimental import pallas as pl


def kernel(n_id, last_update):
    raise NotImplementedError("write your pallas kernel here")


# speedup vs baseline: 43.6282x; 43.6282x over previous
import functools

import jax
import jax.numpy as jnp
from jax import lax
from jax.experimental import pallas as pl
from jax.experimental.pallas import tpu as pltpu
from jax.experimental.pallas import tpu_sc as plsc

N = 3200000
NUM_ROWS = 100000
NW = 32
PAD_N = 3276800
PER_W = PAD_N // NW
G = 10240
NG = PER_W // G

_mesh = plsc.VectorSubcoreMesh(core_axis_name="c", subcore_axis_name="s")


@functools.partial(
    pl.kernel,
    mesh=_mesh,
    out_type=(
        jax.ShapeDtypeStruct((PAD_N,), jnp.int32),
        jax.ShapeDtypeStruct((PAD_N,), jnp.int32),
    ),
    scratch_types=[
        pltpu.VMEM((G,), jnp.int32),
        pltpu.VMEM((G,), jnp.int32),
        pltpu.VMEM((G,), jnp.int32),
        pltpu.SemaphoreType.DMA,
    ],
)
def _sc_gather(idx_hbm, lo_hbm, hi_hbm, out_lo_hbm, out_hi_hbm,
               idx_v, lo_v, hi_v, sem):
    wid = lax.axis_index("s") * 2 + lax.axis_index("c")
    base = wid * jnp.int32(PER_W)

    def body(g, carry):
        off = base + g * jnp.int32(G)
        pltpu.sync_copy(idx_hbm.at[pl.ds(off, G)], idx_v)
        pltpu.async_copy(lo_hbm.at[idx_v], lo_v, sem)
        pltpu.async_copy(hi_hbm.at[idx_v], hi_v, sem).wait()
        pltpu.make_async_copy(lo_hbm.at[idx_v], lo_v, sem).wait()
        pltpu.sync_copy(lo_v, out_lo_hbm.at[pl.ds(off, G)])
        pltpu.sync_copy(hi_v, out_hi_hbm.at[pl.ds(off, G)])
        return carry

    lax.fori_loop(jnp.int32(0), jnp.int32(NG), body, 0)


def kernel(n_id, last_update):
    idx32 = n_id.astype(jnp.int32)
    idx_pad = jnp.concatenate(
        [idx32, jnp.zeros((PAD_N - N,), jnp.int32)])
    pairs = lax.bitcast_convert_type(last_update, jnp.int32)
    lo_plane = pairs[:, 0]
    hi_plane = pairs[:, 1]
    out_lo, out_hi = _sc_gather(idx_pad, lo_plane, hi_plane)
    out_pairs = jnp.stack([out_lo[:N], out_hi[:N]], axis=-1)
    return lax.bitcast_convert_type(out_pairs, jnp.int64)


# speedup vs baseline: 102.9999x; 2.3609x over previous
import functools

import jax
import jax.numpy as jnp
from jax import lax
from jax.experimental import pallas as pl
from jax.experimental.pallas import tpu as pltpu
from jax.experimental.pallas import tpu_sc as plsc

N = 3200000
NUM_ROWS = 100000
NT = 16
PAD_N = 3276800
PER_T = PAD_N // NT
G = 8192
NG = PER_T // G
L = 16

_mesh = plsc.VectorSubcoreMesh(core_axis_name="c", subcore_axis_name="s")


@functools.partial(
    pl.kernel,
    mesh=_mesh,
    compiler_params=pltpu.CompilerParams(needs_layout_passes=False),
    out_type=(
        jax.ShapeDtypeStruct((PAD_N,), jnp.int32),
        jax.ShapeDtypeStruct((PAD_N,), jnp.int32),
    ),
    scratch_types=[
        pltpu.VMEM((NUM_ROWS,), jnp.int32),
        pltpu.VMEM((G,), jnp.int32),
        pltpu.VMEM((G,), jnp.int32),
        pltpu.SemaphoreType.DMA,
    ],
)
def _sc_gather(idx_hbm, lo_hbm, hi_hbm, out_lo_hbm, out_hi_hbm,
               plane_v, idx_v, vals_v, sem):
    wid = lax.axis_index("s") * 2 + lax.axis_index("c")
    is_lo = wid < jnp.int32(NT)
    slot = lax.rem(wid, jnp.int32(NT))
    base = slot * jnp.int32(PER_T)

    @pl.when(is_lo)
    def _():
        pltpu.sync_copy(lo_hbm, plane_v)

    @pl.when(jnp.logical_not(is_lo))
    def _():
        pltpu.sync_copy(hi_hbm, plane_v)

    def group(g, carry):
        off = base + g * jnp.int32(G)
        pltpu.sync_copy(idx_hbm.at[pl.ds(off, G)], idx_v)

        def gbody(i, c):
            o = i * jnp.int32(L)
            ids = idx_v[pl.ds(o, L)]
            vals_v[pl.ds(o, L)] = plsc.load_gather(plane_v, [ids])
            return c

        lax.fori_loop(jnp.int32(0), jnp.int32(G // L), gbody, 0)

        @pl.when(is_lo)
        def _():
            pltpu.sync_copy(vals_v, out_lo_hbm.at[pl.ds(off, G)])

        @pl.when(jnp.logical_not(is_lo))
        def _():
            pltpu.sync_copy(vals_v, out_hi_hbm.at[pl.ds(off, G)])

        return carry

    lax.fori_loop(jnp.int32(0), jnp.int32(NG), group, 0)


def kernel(n_id, last_update):
    idx32 = n_id.astype(jnp.int32)
    idx_pad = jnp.concatenate(
        [idx32, jnp.zeros((PAD_N - N,), jnp.int32)])
    pairs = lax.bitcast_convert_type(last_update, jnp.int32)
    lo_plane = pairs[:, 0]
    hi_plane = pairs[:, 1]
    out_lo, out_hi = _sc_gather(idx_pad, lo_plane, hi_plane)
    out_pairs = jnp.stack([out_lo[:N], out_hi[:N]], axis=-1)
    return lax.bitcast_convert_type(out_pairs, jnp.int64)


# speedup vs baseline: 114.6885x; 1.1135x over previous
import functools

import jax
import jax.numpy as jnp
from jax import lax
from jax.experimental import pallas as pl
from jax.experimental.pallas import tpu as pltpu
from jax.experimental.pallas import tpu_sc as plsc

N = 3200000
NUM_ROWS = 100000
NT = 16
PAD_N = 3276800
PER_T = PAD_N // NT
G = 8192
NG = PER_T // G
L = 16

_mesh = plsc.VectorSubcoreMesh(core_axis_name="c", subcore_axis_name="s")


@functools.partial(
    pl.kernel,
    mesh=_mesh,
    compiler_params=pltpu.CompilerParams(needs_layout_passes=False),
    out_type=(
        jax.ShapeDtypeStruct((PAD_N,), jnp.int32),
        jax.ShapeDtypeStruct((PAD_N,), jnp.int32),
    ),
    scratch_types=[
        pltpu.VMEM((NUM_ROWS,), jnp.int32),
        pltpu.VMEM((G,), jnp.int32),
        pltpu.VMEM((G,), jnp.int32),
        pltpu.SemaphoreType.DMA,
    ],
)
def _sc_gather(idx_hbm, lo_hbm, hi_hbm, out_lo_hbm, out_hi_hbm,
               plane_v, idx_v, vals_v, sem):
    wid = lax.axis_index("s") * 2 + lax.axis_index("c")
    is_lo = wid < jnp.int32(NT)
    slot = lax.rem(wid, jnp.int32(NT))
    base = slot * jnp.int32(PER_T)

    @pl.when(is_lo)
    def _():
        pltpu.sync_copy(lo_hbm, plane_v)

    @pl.when(jnp.logical_not(is_lo))
    def _():
        pltpu.sync_copy(hi_hbm, plane_v)

    def group(g, carry):
        off = base + g * jnp.int32(G)
        pltpu.sync_copy(idx_hbm.at[pl.ds(off, G)], idx_v)

        @plsc.parallel_loop(jnp.int32(0), jnp.int32(G), step=jnp.int32(L),
                            unroll=8)
        def gbody(i):
            ids = idx_v[pl.ds(i, L)]
            vals_v[pl.ds(i, L)] = plsc.load_gather(plane_v, [ids])

        @pl.when(is_lo)
        def _():
            pltpu.sync_copy(vals_v, out_lo_hbm.at[pl.ds(off, G)])

        @pl.when(jnp.logical_not(is_lo))
        def _():
            pltpu.sync_copy(vals_v, out_hi_hbm.at[pl.ds(off, G)])

        return carry

    lax.fori_loop(jnp.int32(0), jnp.int32(NG), group, 0)


def kernel(n_id, last_update):
    idx32 = n_id.astype(jnp.int32)
    idx_pad = jnp.concatenate(
        [idx32, jnp.zeros((PAD_N - N,), jnp.int32)])
    pairs = lax.bitcast_convert_type(last_update, jnp.int32)
    lo_plane = pairs[:, 0]
    hi_plane = pairs[:, 1]
    out_lo, out_hi = _sc_gather(idx_pad, lo_plane, hi_plane)
    out_pairs = jnp.stack([out_lo[:N], out_hi[:N]], axis=-1)
    return lax.bitcast_convert_type(out_pairs, jnp.int64)
